# baseline (speedup 1.0000x reference)
import functools

import numpy as np
import jax
import jax.numpy as jnp
from jax import lax
from jax.experimental import pallas as pl
from jax.experimental.pallas import tpu as pltpu
from jax.experimental.pallas import tpu_sc as plsc

B = 4096
V = 100000
C = 64
T = 50
P = 3 * T
D = P * C

NC, NS = 2, 16
NW = NC * NS
SPC = 2
ROWS = SPC * T
IDXR = B // SPC
NCHUNK = IDXR // NW
_NBUF = 4

BBA = 512


def _pos_encoding() -> np.ndarray:
    half = C // 2
    positions = np.arange(P)[:, np.newaxis]
    dims = np.arange(half)[np.newaxis, :] / half
    rates = 1.0 / 10000 ** dims
    rads = positions * rates
    return np.concatenate([np.sin(rads), np.cos(rads)], axis=-1).astype(np.float32)


_POS = _pos_encoding()
_POS_FLAT = _POS.reshape(1, D)
_POS_I = _POS[0::3].reshape(1, T * C)


def _sc_body(table_hbm, idx_hbm, g_hbm, idx_v,
             buf0, buf1, buf2, buf3,
             gs0, gs1, gs2, gs3, os0, os1, os2, os3):
    c = lax.axis_index("c")
    s = lax.axis_index("s")
    wid = s * NC + c
    row0 = wid * NCHUNK

    pltpu.sync_copy(idx_hbm.at[pl.ds(row0, NCHUNK)], idx_v)

    bufs = (buf0, buf1, buf2, buf3)
    gsems = (gs0, gs1, gs2, gs3)
    osems = (os0, os1, os2, os3)

    for j in range(3):
        pltpu.async_copy(table_hbm.at[idx_v.at[j]], bufs[j], gsems[j])

    for j in range(NCHUNK):
        b = j % _NBUF
        pltpu.make_async_copy(
            table_hbm.at[idx_v.at[j]], bufs[b], gsems[b]).wait()
        pltpu.async_copy(bufs[b], g_hbm.at[row0 + j], osems[b])
        nj = j + 3
        if nj < NCHUNK:
            bn = nj % _NBUF
            if nj >= _NBUF:
                pltpu.make_async_copy(
                    bufs[bn], g_hbm.at[row0 + nj - _NBUF], osems[bn]).wait()
            pltpu.async_copy(table_hbm.at[idx_v.at[nj]], bufs[bn], gsems[bn])

    for j in range(NCHUNK - _NBUF, NCHUNK):
        if j >= 0:
            b = j % _NBUF
            pltpu.make_async_copy(
                bufs[b], g_hbm.at[row0 + j], osems[b]).wait()


@functools.cache
def _sc_gather():
    return pl.kernel(
        _sc_body,
        out_type=jax.ShapeDtypeStruct((IDXR, ROWS, C), jnp.float32),
        mesh=plsc.VectorSubcoreMesh(
            core_axis_name="c", subcore_axis_name="s",
            num_cores=NC, num_subcores=NS),
        scratch_types=[
            pltpu.VMEM((NCHUNK, ROWS), jnp.int32),
            pltpu.VMEM((ROWS, C), jnp.float32),
            pltpu.VMEM((ROWS, C), jnp.float32),
            pltpu.VMEM((ROWS, C), jnp.float32),
            pltpu.VMEM((ROWS, C), jnp.float32),
            pltpu.SemaphoreType.DMA,
            pltpu.SemaphoreType.DMA,
            pltpu.SemaphoreType.DMA,
            pltpu.SemaphoreType.DMA,
            pltpu.SemaphoreType.DMA,
            pltpu.SemaphoreType.DMA,
            pltpu.SemaphoreType.DMA,
            pltpu.SemaphoreType.DMA,
        ],
        compiler_params=pltpu.CompilerParams(use_tc_tiling_on_sc=False),
    )


def _tc_body(et_ref, we_ref, wt_ref, posf_ref, posi_ref, g_ref, out_ref,
             w_scr):
    @pl.when(pl.program_id(0) == 0)
    def _build_w():
        w_scr[...] = jnp.zeros((2 * T, D), jnp.float32)
        for k in range(T):
            w_scr[0:T, (3 * k + 1) * C:(3 * k + 2) * C] = \
                we_ref[:, k * C:(k + 1) * C]
            w_scr[T:2 * T, (3 * k + 2) * C:(3 * k + 3) * C] = \
                wt_ref[:, k * C:(k + 1) * C]

    out_ref[...] = (
        jnp.dot(et_ref[...], w_scr[...], preferred_element_type=jnp.float32)
        + posf_ref[...]
    )
    gp = g_ref[...] + posi_ref[...]
    for k in range(T):
        out_ref[:, 3 * k * C:(3 * k + 1) * C] = gp[:, k * C:(k + 1) * C]


def _tc(et, W_e, W_t, posf, posi, g):
    return pl.pallas_call(
        _tc_body,
        grid=(B // BBA,),
        in_specs=[
            pl.BlockSpec((BBA, 2 * T), lambda i: (i, 0)),
            pl.BlockSpec((T, T * C), lambda i: (0, 0)),
            pl.BlockSpec((T, T * C), lambda i: (0, 0)),
            pl.BlockSpec((1, D), lambda i: (0, 0)),
            pl.BlockSpec((1, T * C), lambda i: (0, 0)),
            pl.BlockSpec((BBA, T * C), lambda i: (i, 0)),
        ],
        out_specs=pl.BlockSpec((BBA, D), lambda i: (i, 0)),
        out_shape=jax.ShapeDtypeStruct((B, D), jnp.float32),
        scratch_shapes=[
            pltpu.VMEM((2 * T, D), jnp.float32),
        ],
    )(et, W_e, W_t, posf, posi, g)


def kernel(x, emb_table, W_e, W_t):
    x3 = x.reshape(B, T, 3)
    et = jnp.concatenate([x3[:, :, 1], x3[:, :, 2]], axis=1)
    idx = x3[:, :, 0].astype(jnp.int32).reshape(IDXR, ROWS)

    g = _sc_gather()(emb_table, idx)

    posf = jnp.asarray(_POS_FLAT)
    posi = jnp.asarray(_POS_I)
    out = _tc(et, W_e, W_t, posf, posi, g.reshape(B, T * C))
    return out.reshape(B, P, C)


# speedup vs baseline: 1.0008x; 1.0008x over previous
import functools

import numpy as np
import jax
import jax.numpy as jnp
from jax import lax
from jax.experimental import pallas as pl
from jax.experimental.pallas import tpu as pltpu
from jax.experimental.pallas import tpu_sc as plsc

B = 4096
V = 100000
C = 64
T = 50
P = 3 * T
D = P * C

NC, NS = 2, 16
NW = NC * NS
SPC = 2
ROWS = SPC * T
IDXR = B // SPC
NCHUNK = IDXR // NW
_NBUF = 6
_PRIME = _NBUF - 1

BBA = 512


def _pos_encoding() -> np.ndarray:
    half = C // 2
    positions = np.arange(P)[:, np.newaxis]
    dims = np.arange(half)[np.newaxis, :] / half
    rates = 1.0 / 10000 ** dims
    rads = positions * rates
    return np.concatenate([np.sin(rads), np.cos(rads)], axis=-1).astype(np.float32)


_POS = _pos_encoding()
_POS_FLAT = _POS.reshape(1, D)
_POS_I = _POS[0::3].reshape(1, T * C)


def _sc_body(table_hbm, idx_hbm, g_hbm, idx_v,
             buf0, buf1, buf2, buf3, buf4, buf5,
             gs0, gs1, gs2, gs3, gs4, gs5,
             os0, os1, os2, os3, os4, os5):
    c = lax.axis_index("c")
    s = lax.axis_index("s")
    wid = s * NC + c
    row0 = wid * NCHUNK

    pltpu.sync_copy(idx_hbm.at[pl.ds(row0, NCHUNK)], idx_v)

    bufs = (buf0, buf1, buf2, buf3, buf4, buf5)
    gsems = (gs0, gs1, gs2, gs3, gs4, gs5)
    osems = (os0, os1, os2, os3, os4, os5)

    for j in range(_PRIME):
        pltpu.async_copy(table_hbm.at[idx_v.at[j]], bufs[j], gsems[j])

    for j in range(NCHUNK):
        b = j % _NBUF
        pltpu.make_async_copy(
            table_hbm.at[idx_v.at[j]], bufs[b], gsems[b]).wait()
        pltpu.async_copy(bufs[b], g_hbm.at[row0 + j], osems[b])
        nj = j + _PRIME
        if nj < NCHUNK:
            bn = nj % _NBUF
            if nj >= _NBUF:
                pltpu.make_async_copy(
                    bufs[bn], g_hbm.at[row0 + nj - _NBUF], osems[bn]).wait()
            pltpu.async_copy(table_hbm.at[idx_v.at[nj]], bufs[bn], gsems[bn])

    for j in range(NCHUNK - _NBUF, NCHUNK):
        if j >= 0:
            b = j % _NBUF
            pltpu.make_async_copy(
                bufs[b], g_hbm.at[row0 + j], osems[b]).wait()


@functools.cache
def _sc_gather():
    return pl.kernel(
        _sc_body,
        out_type=jax.ShapeDtypeStruct((IDXR, ROWS, C), jnp.float32),
        mesh=plsc.VectorSubcoreMesh(
            core_axis_name="c", subcore_axis_name="s",
            num_cores=NC, num_subcores=NS),
        scratch_types=[
            pltpu.VMEM((NCHUNK, ROWS), jnp.int32),
            pltpu.VMEM((ROWS, C), jnp.float32),
            pltpu.VMEM((ROWS, C), jnp.float32),
            pltpu.VMEM((ROWS, C), jnp.float32),
            pltpu.VMEM((ROWS, C), jnp.float32),
            pltpu.VMEM((ROWS, C), jnp.float32),
            pltpu.VMEM((ROWS, C), jnp.float32),
            pltpu.SemaphoreType.DMA,
            pltpu.SemaphoreType.DMA,
            pltpu.SemaphoreType.DMA,
            pltpu.SemaphoreType.DMA,
            pltpu.SemaphoreType.DMA,
            pltpu.SemaphoreType.DMA,
            pltpu.SemaphoreType.DMA,
            pltpu.SemaphoreType.DMA,
            pltpu.SemaphoreType.DMA,
            pltpu.SemaphoreType.DMA,
            pltpu.SemaphoreType.DMA,
            pltpu.SemaphoreType.DMA,
        ],
        compiler_params=pltpu.CompilerParams(use_tc_tiling_on_sc=False),
    )


def _tc_body(et_ref, we_ref, wt_ref, posf_ref, posi_ref, g_ref, out_ref,
             w_scr):
    @pl.when(pl.program_id(0) == 0)
    def _build_w():
        w_scr[...] = jnp.zeros((2 * T, D), jnp.float32)
        for k in range(T):
            w_scr[0:T, (3 * k + 1) * C:(3 * k + 2) * C] = \
                we_ref[:, k * C:(k + 1) * C]
            w_scr[T:2 * T, (3 * k + 2) * C:(3 * k + 3) * C] = \
                wt_ref[:, k * C:(k + 1) * C]

    out_ref[...] = (
        jnp.dot(et_ref[...], w_scr[...], preferred_element_type=jnp.float32)
        + posf_ref[...]
    )
    gp = g_ref[...] + posi_ref[...]
    for k in range(T):
        out_ref[:, 3 * k * C:(3 * k + 1) * C] = gp[:, k * C:(k + 1) * C]


def _tc(et, W_e, W_t, posf, posi, g):
    return pl.pallas_call(
        _tc_body,
        grid=(B // BBA,),
        in_specs=[
            pl.BlockSpec((BBA, 2 * T), lambda i: (i, 0)),
            pl.BlockSpec((T, T * C), lambda i: (0, 0)),
            pl.BlockSpec((T, T * C), lambda i: (0, 0)),
            pl.BlockSpec((1, D), lambda i: (0, 0)),
            pl.BlockSpec((1, T * C), lambda i: (0, 0)),
            pl.BlockSpec((BBA, T * C), lambda i: (i, 0)),
        ],
        out_specs=pl.BlockSpec((BBA, D), lambda i: (i, 0)),
        out_shape=jax.ShapeDtypeStruct((B, D), jnp.float32),
        scratch_shapes=[
            pltpu.VMEM((2 * T, D), jnp.float32),
        ],
    )(et, W_e, W_t, posf, posi, g)


def kernel(x, emb_table, W_e, W_t):
    x3 = x.reshape(B, T, 3)
    et = jnp.concatenate([x3[:, :, 1], x3[:, :, 2]], axis=1)
    idx = x3[:, :, 0].astype(jnp.int32).reshape(IDXR, ROWS)

    g = _sc_gather()(emb_table, idx)

    posf = jnp.asarray(_POS_FLAT)
    posi = jnp.asarray(_POS_I)
    out = _tc(et, W_e, W_t, posf, posi, g.reshape(B, T * C))
    return out.reshape(B, P, C)
